# baseline (speedup 1.0000x reference)
import jax
import jax.numpy as jnp
from jax.experimental import pallas as pl
from jax.experimental.pallas import tpu as pltpu

TOKENS = 256
HIDDEN = 1024
NUM_EXPERTS = 16
FF = 2048
NBUF = 3


def _moe_kernel(x_ref, rl_ref, w1_hbm, w3_hbm, w2_hbm, out_ref,
                w1b, w3b, w2b, sems, i1_ref, i2_ref, g1_ref, g2_ref):
    logits = rl_ref[...]
    mx = jnp.max(logits, axis=-1, keepdims=True)
    ex = jnp.exp(logits - mx)
    p = ex / jnp.sum(ex, axis=-1, keepdims=True)
    eidx = jax.lax.broadcasted_iota(jnp.int32, p.shape, 1)
    m1 = jnp.max(p, axis=-1, keepdims=True)
    i1 = jnp.min(jnp.where(p == m1, eidx, NUM_EXPERTS), axis=-1, keepdims=True)
    p2 = jnp.where(eidx == i1, -1.0, p)
    m2 = jnp.max(p2, axis=-1, keepdims=True)
    i2 = jnp.min(jnp.where(p2 == m2, eidx, NUM_EXPERTS), axis=-1, keepdims=True)
    s = m1 + m2
    i1_ref[...] = i1
    i2_ref[...] = i2
    g1_ref[...] = m1 / s
    g2_ref[...] = m2 / s

    def copies(e, slot):
        return (
            pltpu.make_async_copy(w1_hbm.at[e], w1b.at[slot], sems.at[slot, 0]),
            pltpu.make_async_copy(w3_hbm.at[e], w3b.at[slot], sems.at[slot, 1]),
            pltpu.make_async_copy(w2_hbm.at[e], w2b.at[slot], sems.at[slot, 2]),
        )

    def issue(e, slot):
        for c in copies(e, slot):
            c.start()

    for k in range(NBUF):
        issue(k, k)

    xv = x_ref[...]

    def body(e, _):
        slot = jax.lax.rem(e, NBUF)
        for c in copies(e, slot):
            c.wait()

        h1 = jnp.dot(xv, w1b[slot], preferred_element_type=jnp.float32)
        h3 = jnp.dot(xv, w3b[slot], preferred_element_type=jnp.float32)
        act = ((h1 * jax.lax.logistic(h1)) * h3).astype(jnp.bfloat16)
        y = jnp.dot(act, w2b[slot], preferred_element_type=jnp.float32)

        wcol = (jnp.where(i1_ref[...] == e, g1_ref[...], 0.0)
                + jnp.where(i2_ref[...] == e, g2_ref[...], 0.0))

        @pl.when(e == 0)
        def _():
            out_ref[...] = wcol * y

        @pl.when(e != 0)
        def _():
            out_ref[...] += wcol * y

        @pl.when(e + NBUF < NUM_EXPERTS)
        def _():
            issue(e + NBUF, slot)

        return 0

    jax.lax.fori_loop(0, NUM_EXPERTS, body, 0)


def kernel(x, router_logits, w1, w3, w2):
    return pl.pallas_call(
        _moe_kernel,
        in_specs=[
            pl.BlockSpec(memory_space=pltpu.VMEM),
            pl.BlockSpec(memory_space=pltpu.VMEM),
            pl.BlockSpec(memory_space=pltpu.HBM),
            pl.BlockSpec(memory_space=pltpu.HBM),
            pl.BlockSpec(memory_space=pltpu.HBM),
        ],
        out_specs=pl.BlockSpec(memory_space=pltpu.VMEM),
        out_shape=jax.ShapeDtypeStruct((TOKENS, HIDDEN), jnp.float32),
        scratch_shapes=[
            pltpu.VMEM((NBUF, HIDDEN, FF), jnp.bfloat16),
            pltpu.VMEM((NBUF, HIDDEN, FF), jnp.bfloat16),
            pltpu.VMEM((NBUF, FF, HIDDEN), jnp.bfloat16),
            pltpu.SemaphoreType.DMA((NBUF, 3)),
            pltpu.VMEM((TOKENS, 1), jnp.int32),
            pltpu.VMEM((TOKENS, 1), jnp.int32),
            pltpu.VMEM((TOKENS, 1), jnp.float32),
            pltpu.VMEM((TOKENS, 1), jnp.float32),
        ],
    )(x, router_logits, w1, w3, w2)
